# baseline (speedup 1.0000x reference)
import functools

import jax
import jax.numpy as jnp
from jax import lax
from jax.experimental import pallas as pl
from jax.experimental.pallas import tpu as pltpu
from jax.experimental.pallas import tpu_sc as plsc

N = 10000
E = 320000
D = 128
D2 = D // 2
K = 5

NC = 2
NS = 16
NW = NC * NS
EPT = E // NW
CH = 125
NCH = EPT // CH
NB = 5
LEAD = 2

N_PAD = 10240
RPT = N_PAD // NS

def _sc_mesh():
    return plsc.VectorSubcoreMesh(core_axis_name="c", subcore_axis_name="s",
                                  num_cores=NC, num_subcores=NS)


@functools.cache
def _build_deg_sc():
    return functools.partial(
        pl.kernel,
        out_type=jax.ShapeDtypeStruct((NC, N_PAD), jnp.float32),
        mesh=_sc_mesh(),
        scratch_types=[
            pltpu.VMEM((NCH, CH), jnp.int32),
            pltpu.VMEM((128,), jnp.float32),
            pltpu.VMEM((RPT,), jnp.float32),
            pltpu.VMEM_SHARED((N_PAD,), jnp.float32),
        ],
    )(_deg_sc_body)


def _deg_sc(row_r):
    return _build_deg_sc()(row_r)


def _deg_sc_body(row_hbm, out_hbm, rowv, ones_v, zed_v, acc):
    c = lax.axis_index("c")
    s = lax.axis_index("s")
    wid = c * NS + s

    pltpu.sync_copy(row_hbm.at[wid], rowv)

    @pl.loop(0, 8)
    def _fill(i):
        ones_v[pl.ds(i * 16, 16)] = jnp.full((16,), 1.0, jnp.float32)

    @pl.loop(0, RPT // 16)
    def _zed(i):
        zed_v[pl.ds(i * 16, 16)] = jnp.zeros((16,), jnp.float32)

    pltpu.sync_copy(zed_v, acc.at[pl.ds(s * RPT, RPT)])
    plsc.subcore_barrier()

    @pl.loop(0, NCH)
    def _scatter(j):
        pltpu.sync_copy(ones_v.at[pl.ds(0, CH)], acc.at[rowv.at[j]], add=True)

    plsc.subcore_barrier()
    pltpu.sync_copy(acc.at[pl.ds(s * RPT, RPT)],
                    out_hbm.at[c, pl.ds(s * RPT, RPT)])


@functools.cache
def _build_prop_sc():
    return functools.partial(
        pl.kernel,
        out_type=[jax.ShapeDtypeStruct((N_PAD, D2), jnp.float32)] * 4,
        mesh=_sc_mesh(),
        scratch_types=[
            pltpu.VMEM((NCH, CH), jnp.int32),
            pltpu.VMEM((NCH, CH), jnp.int32),
            [pltpu.VMEM((CH, D2), jnp.float32)] * NB,
            pltpu.VMEM((128, D2), jnp.float32),
            pltpu.VMEM_SHARED((N_PAD, D2), jnp.float32),
            [pltpu.SemaphoreType.DMA] * NB,
            [pltpu.SemaphoreType.DMA] * NB,
        ],
        compiler_params=pltpu.CompilerParams(use_tc_tiling_on_sc=False),
    )(_prop_sc_body)


def _prop_sc(u0, u1, col_r, row_r):
    return _build_prop_sc()(u0, u1, col_r, row_r)


def _prop_sc_body(u0_hbm, u1_hbm, col_hbm, row_hbm,
                  o00_hbm, o01_hbm, o10_hbm, o11_hbm,
                  colv, rowv, gb, zed_v, acc, gsem, ssem):
    c = lax.axis_index("c")
    s = lax.axis_index("s")
    wid = c * NS + s

    pltpu.sync_copy(col_hbm.at[wid], colv)
    pltpu.sync_copy(row_hbm.at[wid], rowv)

    @pl.loop(0, 128)
    def _zed(i):
        for l in range(D2 // 16):
            zed_v[i, pl.ds(l * 16, 16)] = jnp.zeros((16,), jnp.float32)

    @pl.loop(0, RPT // 128)
    def _zcp(i):
        pltpu.sync_copy(zed_v, acc.at[pl.ds(s * RPT + i * 128, 128)])

    def _gather(j, b, u_hbm):
        return pltpu.async_copy(u_hbm.at[colv.at[j]], gb[b], gsem[b])

    def _scat(j, b):
        return pltpu.async_copy(gb[b], acc.at[rowv.at[j]], ssem[b],
                                add=True)

    ng = NCH // NB
    for h, (u_hbm, out_c0, out_c1) in enumerate(((u0_hbm, o00_hbm, o10_hbm),
                                                 (u1_hbm, o01_hbm, o11_hbm))):
        plsc.subcore_barrier()

        for b in range(LEAD):
            _gather(b, b, u_hbm)

        for b in range(NB):
            pltpu.make_async_copy(u_hbm.at[colv.at[b]],
                                  gb[b], gsem[b]).wait()
            _scat(b, b)
            b2 = (b + LEAD) % NB
            if b < NB - LEAD:
                _gather(b + LEAD, b2, u_hbm)
            else:
                pltpu.make_async_copy(
                    gb[b2], acc.at[rowv.at[b - (NB - LEAD)]],
                    ssem[b2]).wait()
                _gather(b + LEAD, b2, u_hbm)

        @pl.loop(1, ng)
        def _grp(g):
            for b in range(NB):
                j = g * NB + b
                pltpu.make_async_copy(u_hbm.at[colv.at[j]],
                                      gb[b], gsem[b]).wait()
                _scat(j, b)

                b2 = (b + LEAD) % NB

                def _refill(j=j, b2=b2):
                    pltpu.make_async_copy(
                        gb[b2], acc.at[rowv.at[j - (NB - LEAD)]],
                        ssem[b2]).wait()
                    _gather(j + LEAD, b2, u_hbm)

                if b < NB - LEAD:
                    _refill()
                else:
                    pl.when(g < ng - 1)(_refill)

        for j in range(NCH - NB, NCH):
            pltpu.make_async_copy(gb[j % NB], acc.at[rowv.at[j]],
                                  ssem[j % NB]).wait()

        plsc.subcore_barrier()

        @pl.when(c == 0)
        def _wr0():
            pltpu.sync_copy(acc.at[pl.ds(s * RPT, RPT)],
                            out_c0.at[pl.ds(s * RPT, RPT)])

        @pl.when(c == 1)
        def _wr1():
            pltpu.sync_copy(acc.at[pl.ds(s * RPT, RPT)],
                            out_c1.at[pl.ds(s * RPT, RPT)])

        if h == 0:
            @pl.loop(0, RPT // 128)
            def _rz(i):
                pltpu.sync_copy(zed_v, acc.at[pl.ds(s * RPT + i * 128, 128)])


_BR = 2048
_GRID = N_PAD // _BR

NH = N_PAD // 2
_BRH = _BR // 2

_row_spec = pl.BlockSpec((_BR, D), lambda i: (i, 0))
_w_spec = pl.BlockSpec((D, D), lambda i: (0, 0))
_pk_spec = pl.BlockSpec((_BRH, D), lambda i: (i, 0))

_f32 = jnp.float32


def _tc_prep_body(dp2_ref, dinvp_ref):
    deg = dp2_ref[0] + dp2_ref[1]
    dinv = jnp.where(deg > 0.0, lax.rsqrt(jnp.maximum(deg, 1e-12)), 0.0)
    lane = lax.broadcasted_iota(jnp.int32, (_BRH, D), 1)
    dinvp_ref[...] = jnp.where(lane < D2, dinv[:, 0:1], dinv[:, 1:2])


_tc_prep = pl.pallas_call(
    _tc_prep_body,
    grid=(_GRID,),
    in_specs=[pl.BlockSpec((NC, _BRH, 2), lambda i: (0, i, 0))],
    out_specs=_pk_spec,
    out_shape=jax.ShapeDtypeStruct((NH, D), _f32),
)


_e_spec = pl.BlockSpec((2 * D, D), lambda i: (0, 0))
_bp_spec = pl.BlockSpec((1, D), lambda i: (0, 0))


def _pk_mm(t0p, t1p, e0_ref, e1_ref):
    tcat = jnp.concatenate([t0p, t1p], axis=1)
    a0 = jnp.dot(tcat, e0_ref[...], preferred_element_type=_f32)
    a1 = jnp.dot(tcat, e1_ref[...], preferred_element_type=_f32)
    return a0, a1


def _make_tc_start(first):
    def body(*refs):
        if first:
            h0_ref, h1_ref, dinvp_ref, e0_ref, e1_ref, *outs = refs
            h0p, h1p = h0_ref[...], h1_ref[...]
        else:
            (a0_ref, a1_ref, b0_ref, b1_ref, dinvp_ref, e0_ref, e1_ref,
             *outs) = refs
            h0p = jnp.maximum(a0_ref[...] + b0_ref[...], 0.0)
            h1p = jnp.maximum(a1_ref[...] + b1_ref[...], 0.0)
        h0p_ref, h1p_ref, u0p_ref, u1p_ref, acc0_ref, acc1_ref = outs
        dinvp = dinvp_ref[...]
        h0p_ref[...] = h0p
        h1p_ref[...] = h1p
        u0p_ref[...] = dinvp * h0p
        u1p_ref[...] = dinvp * h1p
        a0, a1 = _pk_mm(h0p, h1p, e0_ref, e1_ref)
        acc0_ref[...] = a0
        acc1_ref[...] = a1

    pk_shape = jax.ShapeDtypeStruct((NH, D), _f32)
    if first:
        in_specs = [_pk_spec, _pk_spec, _pk_spec, _e_spec, _e_spec]
    else:
        in_specs = [_pk_spec, _pk_spec, _bp_spec, _bp_spec,
                    _pk_spec, _e_spec, _e_spec]
    out_shape = [pk_shape] * 6
    out_specs = [_pk_spec] * 6
    return pl.pallas_call(body, grid=(_GRID,), in_specs=in_specs,
                          out_specs=out_specs, out_shape=out_shape)


_tc_start_first = _make_tc_start(True)
_tc_start_next = _make_tc_start(False)


def _make_tc_step(with_prev, emit_u, emit_tx=True):
    alpha = 2.0 if with_prev else 1.0

    def body(*refs):
        if with_prev:
            (p00_ref, p01_ref, p10_ref, p11_ref, tp0_ref, tp1_ref,
             dinvp_ref, e0_ref, e1_ref, a0_ref, a1_ref, *outs) = refs
        else:
            (p00_ref, p01_ref, p10_ref, p11_ref,
             dinvp_ref, e0_ref, e1_ref, a0_ref, a1_ref, *outs) = refs
        dinvp = dinvp_ref[...]
        t0p = -alpha * dinvp * (p00_ref[...] + p10_ref[...])
        t1p = -alpha * dinvp * (p01_ref[...] + p11_ref[...])
        if with_prev:
            t0p = t0p - tp0_ref[...]
            t1p = t1p - tp1_ref[...]
        outs = list(outs)
        if emit_tx:
            outs.pop(0)[...] = t0p
            outs.pop(0)[...] = t1p
        if emit_u:
            outs.pop(0)[...] = dinvp * t0p
            outs.pop(0)[...] = dinvp * t1p
        a0, a1 = _pk_mm(t0p, t1p, e0_ref, e1_ref)
        outs.pop(0)[...] = a0_ref[...] + a0
        outs.pop(0)[...] = a1_ref[...] + a1

    pk_shape = jax.ShapeDtypeStruct((NH, D), _f32)
    in_specs = [_pk_spec] * 4
    if with_prev:
        in_specs += [_pk_spec, _pk_spec]
    in_specs += [_pk_spec, _e_spec, _e_spec, _pk_spec, _pk_spec]
    n_pk = (2 if emit_tx else 0) + (2 if emit_u else 0) + 2
    out_shape = [pk_shape] * n_pk
    out_specs = [_pk_spec] * n_pk
    return pl.pallas_call(body, grid=(_GRID,), in_specs=in_specs,
                          out_specs=out_specs, out_shape=out_shape)


_tc_step_k1 = _make_tc_step(False, True)
_tc_step_mid = _make_tc_step(True, True)
_tc_step_last = _make_tc_step(True, False, emit_tx=False)


def _tc_head_body(a0_ref, a1_ref, b0_ref, b1_ref, el_ref, bl_ref, y_ref):
    tcat = jnp.concatenate([a0_ref[...] + b0_ref[...],
                            a1_ref[...] + b1_ref[...]], axis=1)
    y_ref[...] = jnp.dot(tcat, el_ref[...],
                         preferred_element_type=_f32) + bl_ref[0, 0]


_tc_head = pl.pallas_call(
    _tc_head_body,
    grid=(_GRID,),
    in_specs=[_pk_spec, _pk_spec, _bp_spec, _bp_spec,
              pl.BlockSpec((2 * D, 2), lambda i: (0, 0)),
              pl.BlockSpec((1, 1), lambda i: (0, 0))],
    out_specs=pl.BlockSpec((_BRH, 2), lambda i: (i, 0)),
    out_shape=jax.ShapeDtypeStruct((NH, 2), _f32),
)


def _blkdiag2(m):
    z = jnp.zeros_like(m)
    return jnp.concatenate([jnp.concatenate([m, z], 1),
                            jnp.concatenate([z, m], 1)], 0)


def _expand_w(w):
    e0 = jnp.concatenate([_blkdiag2(w[:D2, :D2]), _blkdiag2(w[D2:, :D2])], 0)
    e1 = jnp.concatenate([_blkdiag2(w[:D2, D2:]), _blkdiag2(w[D2:, D2:])], 0)
    return e0, e1


def _pack_vec(v):
    return (jnp.concatenate([v[:D2], v[:D2]]).reshape(1, D),
            jnp.concatenate([v[D2:], v[D2:]]).reshape(1, D))


def kernel(x, edge_index, W1, b1, W2, b2, W3, b3, Wl, bl):
    row_r = edge_index[0].reshape(NW, NCH, CH)
    col_r = edge_index[1].reshape(NW, NCH, CH)

    xp = jnp.concatenate([x, jnp.zeros((N_PAD - N, D), jnp.float32)], axis=0)
    x0p = jnp.reshape(xp[:, :D2], (NH, D))
    x1p = jnp.reshape(xp[:, D2:], (NH, D))

    dp = _deg_sc(row_r)
    dinvp = _tc_prep(jnp.reshape(dp, (NC, NH, 2)))

    def _sc_view(a):
        return jnp.reshape(a, (N_PAD, D2))

    def _tc_view(a):
        return jnp.reshape(a, (NH, D))

    a0 = a1 = None
    bprev = None
    for li, (W, b) in enumerate(((W1, b1), (W2, b2), (W3, b3))):
        es = [_expand_w(W[k]) for k in range(K)]
        if li == 0:
            h0p, h1p, u0p, u1p, a0, a1 = _tc_start_first(
                x0p, x1p, dinvp, *es[0])
        else:
            b0p, b1p = _pack_vec(bprev)
            h0p, h1p, u0p, u1p, a0, a1 = _tc_start_next(
                a0, a1, b0p, b1p, dinvp, *es[0])
        txs = [(h0p, h1p)]
        for k in range(1, K):
            ps = _prop_sc(_sc_view(u0p), _sc_view(u1p), col_r, row_r)
            pr = tuple(_tc_view(p) for p in ps)
            if k == 1:
                t0p, t1p, u0p, u1p, a0, a1 = _tc_step_k1(
                    *pr, dinvp, *es[k], a0, a1)
            elif k < K - 1:
                t0p, t1p, u0p, u1p, a0, a1 = _tc_step_mid(
                    *pr, *txs[k - 2], dinvp, *es[k], a0, a1)
            else:
                a0, a1 = _tc_step_last(*pr, *txs[k - 2], dinvp, *es[k],
                                       a0, a1)
            txs.append((t0p, t1p))
        bprev = b

    el = jnp.concatenate([_blkdiag2(Wl[:D2]), _blkdiag2(Wl[D2:])], 0)
    b30p, b31p = _pack_vec(b3)
    yp = _tc_head(a0, a1, b30p, b31p, el, bl.reshape(1, 1))
    return jnp.reshape(yp, (N_PAD, 1))[:N]


# speedup vs baseline: 1.1513x; 1.1513x over previous
import functools

import jax
import jax.numpy as jnp
from jax import lax
from jax.experimental import pallas as pl
from jax.experimental.pallas import tpu as pltpu
from jax.experimental.pallas import tpu_sc as plsc

N = 10000
E = 320000
D = 128
D2 = D // 2
K = 5

NC = 2
NS = 16
NW = NC * NS
EPT = E // NW
CH = 125
NCH = EPT // CH
NB = 5
LEAD = 3

N_PAD = 10240
RPT = N_PAD // NS

def _sc_mesh():
    return plsc.VectorSubcoreMesh(core_axis_name="c", subcore_axis_name="s",
                                  num_cores=NC, num_subcores=NS)


@functools.cache
def _build_deg_sc():
    return functools.partial(
        pl.kernel,
        out_type=jax.ShapeDtypeStruct((NC, N_PAD), jnp.float32),
        mesh=_sc_mesh(),
        scratch_types=[
            pltpu.VMEM((NCH, CH), jnp.int32),
            pltpu.VMEM((128,), jnp.float32),
            pltpu.VMEM((RPT,), jnp.float32),
            pltpu.VMEM_SHARED((N_PAD,), jnp.float32),
            [pltpu.SemaphoreType.DMA] * 4,
        ],
    )(_deg_sc_body)


def _deg_sc(row_r):
    return _build_deg_sc()(row_r)


def _deg_sc_body(row_hbm, out_hbm, rowv, ones_v, zed_v, acc, dsem):
    c = lax.axis_index("c")
    s = lax.axis_index("s")
    wid = c * NS + s

    pltpu.sync_copy(row_hbm.at[wid], rowv)

    @pl.loop(0, 8)
    def _fill(i):
        ones_v[pl.ds(i * 16, 16)] = jnp.full((16,), 1.0, jnp.float32)

    @pl.loop(0, RPT // 16)
    def _zed(i):
        zed_v[pl.ds(i * 16, 16)] = jnp.zeros((16,), jnp.float32)

    pltpu.sync_copy(zed_v, acc.at[pl.ds(s * RPT, RPT)])
    plsc.subcore_barrier()

    @pl.loop(0, NCH // 4)
    def _scatter(g):
        for b in range(4):
            j = g * 4 + b

            def _wait_prev(j=j, b=b):
                pltpu.make_async_copy(ones_v.at[pl.ds(0, CH)],
                                      acc.at[rowv.at[j - 4]],
                                      dsem[b]).wait()

            pl.when(g > 0)(_wait_prev)
            pltpu.async_copy(ones_v.at[pl.ds(0, CH)],
                             acc.at[rowv.at[j]], dsem[b], add=True)

    for j in range(NCH - 4, NCH):
        pltpu.make_async_copy(ones_v.at[pl.ds(0, CH)],
                              acc.at[rowv.at[j]], dsem[j % 4]).wait()

    plsc.subcore_barrier()
    pltpu.sync_copy(acc.at[pl.ds(s * RPT, RPT)],
                    out_hbm.at[c, pl.ds(s * RPT, RPT)])


@functools.cache
def _build_prop_sc():
    return functools.partial(
        pl.kernel,
        out_type=[jax.ShapeDtypeStruct((N_PAD, D2), jnp.float32)] * 4,
        mesh=_sc_mesh(),
        scratch_types=[
            pltpu.VMEM((NCH, CH), jnp.int32),
            pltpu.VMEM((NCH, CH), jnp.int32),
            [pltpu.VMEM((CH, D2), jnp.float32)] * NB,
            pltpu.VMEM((128, D2), jnp.float32),
            pltpu.VMEM_SHARED((N_PAD, D2), jnp.float32),
            [pltpu.SemaphoreType.DMA] * NB,
            [pltpu.SemaphoreType.DMA] * NB,
        ],
        compiler_params=pltpu.CompilerParams(use_tc_tiling_on_sc=False),
    )(_prop_sc_body)


def _prop_sc(u0, u1, col_r, row_r):
    return _build_prop_sc()(u0, u1, col_r, row_r)


def _prop_sc_body(u0_hbm, u1_hbm, col_hbm, row_hbm,
                  o00_hbm, o01_hbm, o10_hbm, o11_hbm,
                  colv, rowv, gb, zed_v, acc, gsem, ssem):
    c = lax.axis_index("c")
    s = lax.axis_index("s")
    wid = c * NS + s

    pltpu.sync_copy(col_hbm.at[wid], colv)
    pltpu.sync_copy(row_hbm.at[wid], rowv)

    @pl.loop(0, 128)
    def _zed(i):
        for l in range(D2 // 16):
            zed_v[i, pl.ds(l * 16, 16)] = jnp.zeros((16,), jnp.float32)

    @pl.loop(0, RPT // 128)
    def _zcp(i):
        pltpu.sync_copy(zed_v, acc.at[pl.ds(s * RPT + i * 128, 128)])

    def _gather(j, b, u_hbm):
        return pltpu.async_copy(u_hbm.at[colv.at[j]], gb[b], gsem[b])

    def _scat(j, b):
        return pltpu.async_copy(gb[b], acc.at[rowv.at[j]], ssem[b],
                                add=True)

    ng = NCH // NB
    for h, (u_hbm, out_c0, out_c1) in enumerate(((u0_hbm, o00_hbm, o10_hbm),
                                                 (u1_hbm, o01_hbm, o11_hbm))):
        plsc.subcore_barrier()

        for b in range(LEAD):
            _gather(b, b, u_hbm)

        for b in range(NB):
            pltpu.make_async_copy(u_hbm.at[colv.at[b]],
                                  gb[b], gsem[b]).wait()
            _scat(b, b)
            b2 = (b + LEAD) % NB
            if b < NB - LEAD:
                _gather(b + LEAD, b2, u_hbm)
            else:
                pltpu.make_async_copy(
                    gb[b2], acc.at[rowv.at[b - (NB - LEAD)]],
                    ssem[b2]).wait()
                _gather(b + LEAD, b2, u_hbm)

        @pl.loop(1, ng)
        def _grp(g):
            for b in range(NB):
                j = g * NB + b
                pltpu.make_async_copy(u_hbm.at[colv.at[j]],
                                      gb[b], gsem[b]).wait()
                _scat(j, b)

                b2 = (b + LEAD) % NB

                def _refill(j=j, b2=b2):
                    pltpu.make_async_copy(
                        gb[b2], acc.at[rowv.at[j - (NB - LEAD)]],
                        ssem[b2]).wait()
                    _gather(j + LEAD, b2, u_hbm)

                if b < NB - LEAD:
                    _refill()
                else:
                    pl.when(g < ng - 1)(_refill)

        for j in range(NCH - NB, NCH):
            pltpu.make_async_copy(gb[j % NB], acc.at[rowv.at[j]],
                                  ssem[j % NB]).wait()

        plsc.subcore_barrier()

        @pl.when(c == 0)
        def _wr0():
            pltpu.sync_copy(acc.at[pl.ds(s * RPT, RPT)],
                            out_c0.at[pl.ds(s * RPT, RPT)])

        @pl.when(c == 1)
        def _wr1():
            pltpu.sync_copy(acc.at[pl.ds(s * RPT, RPT)],
                            out_c1.at[pl.ds(s * RPT, RPT)])

        if h == 0:
            @pl.loop(0, RPT // 128)
            def _rz(i):
                pltpu.sync_copy(zed_v, acc.at[pl.ds(s * RPT + i * 128, 128)])


_BR = 2048
_GRID = N_PAD // _BR

NH = N_PAD // 2
_BRH = _BR // 2

_row_spec = pl.BlockSpec((_BR, D), lambda i: (i, 0))
_w_spec = pl.BlockSpec((D, D), lambda i: (0, 0))
_pk_spec = pl.BlockSpec((_BRH, D), lambda i: (i, 0))

_f32 = jnp.float32


def _tc_prep_body(dp2_ref, dinvp_ref):
    deg = dp2_ref[0] + dp2_ref[1]
    dinv = jnp.where(deg > 0.0, lax.rsqrt(jnp.maximum(deg, 1e-12)), 0.0)
    lane = lax.broadcasted_iota(jnp.int32, (_BRH, D), 1)
    dinvp_ref[...] = jnp.where(lane < D2, dinv[:, 0:1], dinv[:, 1:2])


_tc_prep = pl.pallas_call(
    _tc_prep_body,
    grid=(_GRID,),
    in_specs=[pl.BlockSpec((NC, _BRH, 2), lambda i: (0, i, 0))],
    out_specs=_pk_spec,
    out_shape=jax.ShapeDtypeStruct((NH, D), _f32),
)


_e_spec = pl.BlockSpec((2 * D, D), lambda i: (0, 0))
_bp_spec = pl.BlockSpec((1, D), lambda i: (0, 0))


def _pk_mm(t0p, t1p, e0_ref, e1_ref):
    tcat = jnp.concatenate([t0p, t1p], axis=1)
    a0 = jnp.dot(tcat, e0_ref[...], preferred_element_type=_f32)
    a1 = jnp.dot(tcat, e1_ref[...], preferred_element_type=_f32)
    return a0, a1


def _make_tc_start(first):
    def body(*refs):
        if first:
            h0_ref, h1_ref, dinvp_ref, e0_ref, e1_ref, *outs = refs
            h0p, h1p = h0_ref[...], h1_ref[...]
        else:
            (a0_ref, a1_ref, b0_ref, b1_ref, dinvp_ref, e0_ref, e1_ref,
             *outs) = refs
            h0p = jnp.maximum(a0_ref[...] + b0_ref[...], 0.0)
            h1p = jnp.maximum(a1_ref[...] + b1_ref[...], 0.0)
        h0p_ref, h1p_ref, u0p_ref, u1p_ref, acc0_ref, acc1_ref = outs
        dinvp = dinvp_ref[...]
        h0p_ref[...] = h0p
        h1p_ref[...] = h1p
        u0p_ref[...] = dinvp * h0p
        u1p_ref[...] = dinvp * h1p
        a0, a1 = _pk_mm(h0p, h1p, e0_ref, e1_ref)
        acc0_ref[...] = a0
        acc1_ref[...] = a1

    pk_shape = jax.ShapeDtypeStruct((NH, D), _f32)
    if first:
        in_specs = [_pk_spec, _pk_spec, _pk_spec, _e_spec, _e_spec]
    else:
        in_specs = [_pk_spec, _pk_spec, _bp_spec, _bp_spec,
                    _pk_spec, _e_spec, _e_spec]
    out_shape = [pk_shape] * 6
    out_specs = [_pk_spec] * 6
    return pl.pallas_call(body, grid=(_GRID,), in_specs=in_specs,
                          out_specs=out_specs, out_shape=out_shape)


_tc_start_first = _make_tc_start(True)
_tc_start_next = _make_tc_start(False)


def _make_tc_step(with_prev, emit_u, emit_tx=True):
    alpha = 2.0 if with_prev else 1.0

    def body(*refs):
        if with_prev:
            (p00_ref, p01_ref, p10_ref, p11_ref, tp0_ref, tp1_ref,
             dinvp_ref, e0_ref, e1_ref, a0_ref, a1_ref, *outs) = refs
        else:
            (p00_ref, p01_ref, p10_ref, p11_ref,
             dinvp_ref, e0_ref, e1_ref, a0_ref, a1_ref, *outs) = refs
        dinvp = dinvp_ref[...]
        t0p = -alpha * dinvp * (p00_ref[...] + p10_ref[...])
        t1p = -alpha * dinvp * (p01_ref[...] + p11_ref[...])
        if with_prev:
            t0p = t0p - tp0_ref[...]
            t1p = t1p - tp1_ref[...]
        outs = list(outs)
        if emit_tx:
            outs.pop(0)[...] = t0p
            outs.pop(0)[...] = t1p
        if emit_u:
            outs.pop(0)[...] = dinvp * t0p
            outs.pop(0)[...] = dinvp * t1p
        a0, a1 = _pk_mm(t0p, t1p, e0_ref, e1_ref)
        outs.pop(0)[...] = a0_ref[...] + a0
        outs.pop(0)[...] = a1_ref[...] + a1

    pk_shape = jax.ShapeDtypeStruct((NH, D), _f32)
    in_specs = [_pk_spec] * 4
    if with_prev:
        in_specs += [_pk_spec, _pk_spec]
    in_specs += [_pk_spec, _e_spec, _e_spec, _pk_spec, _pk_spec]
    n_pk = (2 if emit_tx else 0) + (2 if emit_u else 0) + 2
    out_shape = [pk_shape] * n_pk
    out_specs = [_pk_spec] * n_pk
    return pl.pallas_call(body, grid=(_GRID,), in_specs=in_specs,
                          out_specs=out_specs, out_shape=out_shape)


_tc_step_k1 = _make_tc_step(False, True)
_tc_step_mid = _make_tc_step(True, True)
_tc_step_last = _make_tc_step(True, False, emit_tx=False)


def _tc_head_body(a0_ref, a1_ref, b0_ref, b1_ref, el_ref, bl_ref, y_ref):
    tcat = jnp.concatenate([a0_ref[...] + b0_ref[...],
                            a1_ref[...] + b1_ref[...]], axis=1)
    y_ref[...] = jnp.dot(tcat, el_ref[...],
                         preferred_element_type=_f32) + bl_ref[0, 0]


_tc_head = pl.pallas_call(
    _tc_head_body,
    grid=(_GRID,),
    in_specs=[_pk_spec, _pk_spec, _bp_spec, _bp_spec,
              pl.BlockSpec((2 * D, 2), lambda i: (0, 0)),
              pl.BlockSpec((1, 1), lambda i: (0, 0))],
    out_specs=pl.BlockSpec((_BRH, 2), lambda i: (i, 0)),
    out_shape=jax.ShapeDtypeStruct((NH, 2), _f32),
)


def _blkdiag2(m):
    z = jnp.zeros_like(m)
    return jnp.concatenate([jnp.concatenate([m, z], 1),
                            jnp.concatenate([z, m], 1)], 0)


def _expand_w(w):
    e0 = jnp.concatenate([_blkdiag2(w[:D2, :D2]), _blkdiag2(w[D2:, :D2])], 0)
    e1 = jnp.concatenate([_blkdiag2(w[:D2, D2:]), _blkdiag2(w[D2:, D2:])], 0)
    return e0, e1


def _pack_vec(v):
    return (jnp.concatenate([v[:D2], v[:D2]]).reshape(1, D),
            jnp.concatenate([v[D2:], v[D2:]]).reshape(1, D))


def kernel(x, edge_index, W1, b1, W2, b2, W3, b3, Wl, bl):
    row_r = edge_index[0].reshape(NW, NCH, CH)
    col_r = edge_index[1].reshape(NW, NCH, CH)

    xp = jnp.concatenate([x, jnp.zeros((N_PAD - N, D), jnp.float32)], axis=0)
    x0p = jnp.reshape(xp[:, :D2], (NH, D))
    x1p = jnp.reshape(xp[:, D2:], (NH, D))

    dp = _deg_sc(row_r)
    dinvp = _tc_prep(jnp.reshape(dp, (NC, NH, 2)))

    def _sc_view(a):
        return jnp.reshape(a, (N_PAD, D2))

    def _tc_view(a):
        return jnp.reshape(a, (NH, D))

    a0 = a1 = None
    bprev = None
    for li, (W, b) in enumerate(((W1, b1), (W2, b2), (W3, b3))):
        es = [_expand_w(W[k]) for k in range(K)]
        if li == 0:
            h0p, h1p, u0p, u1p, a0, a1 = _tc_start_first(
                x0p, x1p, dinvp, *es[0])
        else:
            b0p, b1p = _pack_vec(bprev)
            h0p, h1p, u0p, u1p, a0, a1 = _tc_start_next(
                a0, a1, b0p, b1p, dinvp, *es[0])
        txs = [(h0p, h1p)]
        for k in range(1, K):
            ps = _prop_sc(_sc_view(u0p), _sc_view(u1p), col_r, row_r)
            pr = tuple(_tc_view(p) for p in ps)
            if k == 1:
                t0p, t1p, u0p, u1p, a0, a1 = _tc_step_k1(
                    *pr, dinvp, *es[k], a0, a1)
            elif k < K - 1:
                t0p, t1p, u0p, u1p, a0, a1 = _tc_step_mid(
                    *pr, *txs[k - 2], dinvp, *es[k], a0, a1)
            else:
                a0, a1 = _tc_step_last(*pr, *txs[k - 2], dinvp, *es[k],
                                       a0, a1)
            txs.append((t0p, t1p))
        bprev = b

    el = jnp.concatenate([_blkdiag2(Wl[:D2]), _blkdiag2(Wl[D2:])], 0)
    b30p, b31p = _pack_vec(b3)
    yp = _tc_head(a0, a1, b30p, b31p, el, bl.reshape(1, 1))
    return jnp.reshape(yp, (N_PAD, 1))[:N]


# speedup vs baseline: 1.1606x; 1.0081x over previous
import functools

import jax
import jax.numpy as jnp
from jax import lax
from jax.experimental import pallas as pl
from jax.experimental.pallas import tpu as pltpu
from jax.experimental.pallas import tpu_sc as plsc

N = 10000
E = 320000
D = 128
D2 = D // 2
K = 5

NC = 2
NS = 16
NW = NC * NS
EPT = E // NW
CH = 125
NCH = EPT // CH
NB = 5
LEAD = 3

N_PAD = 10240
RPT = N_PAD // NS

def _sc_mesh():
    return plsc.VectorSubcoreMesh(core_axis_name="c", subcore_axis_name="s",
                                  num_cores=NC, num_subcores=NS)


@functools.cache
def _build_deg_sc():
    return functools.partial(
        pl.kernel,
        out_type=jax.ShapeDtypeStruct((NC, N_PAD), jnp.float32),
        mesh=_sc_mesh(),
        scratch_types=[
            pltpu.VMEM((NCH, CH), jnp.int32),
            pltpu.VMEM((128,), jnp.float32),
            pltpu.VMEM((RPT,), jnp.float32),
            pltpu.VMEM_SHARED((N_PAD,), jnp.float32),
            [pltpu.SemaphoreType.DMA] * 4,
        ],
    )(_deg_sc_body)


def _deg_sc(row_r):
    return _build_deg_sc()(row_r)


def _deg_sc_body(row_hbm, out_hbm, rowv, ones_v, zed_v, acc, dsem):
    c = lax.axis_index("c")
    s = lax.axis_index("s")
    wid = c * NS + s

    pltpu.sync_copy(row_hbm.at[wid], rowv)

    @pl.loop(0, 8)
    def _fill(i):
        ones_v[pl.ds(i * 16, 16)] = jnp.full((16,), 1.0, jnp.float32)

    @pl.loop(0, RPT // 16)
    def _zed(i):
        zed_v[pl.ds(i * 16, 16)] = jnp.zeros((16,), jnp.float32)

    pltpu.sync_copy(zed_v, acc.at[pl.ds(s * RPT, RPT)])
    plsc.subcore_barrier()

    @pl.loop(0, NCH // 4)
    def _scatter(g):
        for b in range(4):
            j = g * 4 + b

            def _wait_prev(j=j, b=b):
                pltpu.make_async_copy(ones_v.at[pl.ds(0, CH)],
                                      acc.at[rowv.at[j - 4]],
                                      dsem[b]).wait()

            pl.when(g > 0)(_wait_prev)
            pltpu.async_copy(ones_v.at[pl.ds(0, CH)],
                             acc.at[rowv.at[j]], dsem[b], add=True)

    for j in range(NCH - 4, NCH):
        pltpu.make_async_copy(ones_v.at[pl.ds(0, CH)],
                              acc.at[rowv.at[j]], dsem[j % 4]).wait()

    plsc.subcore_barrier()
    pltpu.sync_copy(acc.at[pl.ds(s * RPT, RPT)],
                    out_hbm.at[c, pl.ds(s * RPT, RPT)])


@functools.cache
def _build_prop_sc():
    return functools.partial(
        pl.kernel,
        out_type=[jax.ShapeDtypeStruct((N_PAD, D2), jnp.float32)] * 4,
        mesh=_sc_mesh(),
        scratch_types=[
            pltpu.VMEM((NCH, CH), jnp.int32),
            pltpu.VMEM((NCH, CH), jnp.int32),
            [pltpu.VMEM((CH, D2), jnp.float32)] * NB,
            pltpu.VMEM((128, D2), jnp.float32),
            pltpu.VMEM_SHARED((N_PAD, D2), jnp.float32),
            [pltpu.SemaphoreType.DMA] * NB,
            [pltpu.SemaphoreType.DMA] * NB,
        ],
        compiler_params=pltpu.CompilerParams(use_tc_tiling_on_sc=False),
    )(_prop_sc_body)


def _prop_sc(u0, u1, col_r, row_r):
    return _build_prop_sc()(u0, u1, col_r, row_r)


def _prop_sc_body(u0_hbm, u1_hbm, col_hbm, row_hbm,
                  o00_hbm, o01_hbm, o10_hbm, o11_hbm,
                  colv, rowv, gb, zed_v, acc, gsem, ssem):
    c = lax.axis_index("c")
    s = lax.axis_index("s")
    wid = c * NS + s

    pltpu.sync_copy(col_hbm.at[wid], colv)
    pltpu.sync_copy(row_hbm.at[wid], rowv)

    @pl.loop(0, 128)
    def _zed(i):
        for l in range(D2 // 16):
            zed_v[i, pl.ds(l * 16, 16)] = jnp.zeros((16,), jnp.float32)

    @pl.loop(0, RPT // 128)
    def _zcp(i):
        pltpu.sync_copy(zed_v, acc.at[pl.ds(s * RPT + i * 128, 128)])

    def _gather(j, b, u_hbm):
        return pltpu.async_copy(u_hbm.at[colv.at[j]], gb[b], gsem[b])

    def _scat(j, b):
        return pltpu.async_copy(gb[b], acc.at[rowv.at[j]], ssem[b],
                                add=True)

    ng = NCH // NB
    for h, (u_hbm, out_c0, out_c1) in enumerate(((u0_hbm, o00_hbm, o10_hbm),
                                                 (u1_hbm, o01_hbm, o11_hbm))):
        plsc.subcore_barrier()

        for b in range(LEAD):
            _gather(b, b, u_hbm)

        for b in range(NB):
            pltpu.make_async_copy(u_hbm.at[colv.at[b]],
                                  gb[b], gsem[b]).wait()
            _scat(b, b)
            b2 = (b + LEAD) % NB
            if b < NB - LEAD:
                _gather(b + LEAD, b2, u_hbm)
            else:
                pltpu.make_async_copy(
                    gb[b2], acc.at[rowv.at[b - (NB - LEAD)]],
                    ssem[b2]).wait()
                _gather(b + LEAD, b2, u_hbm)

        @pl.loop(1, ng)
        def _grp(g):
            for b in range(NB):
                j = g * NB + b
                pltpu.make_async_copy(u_hbm.at[colv.at[j]],
                                      gb[b], gsem[b]).wait()
                _scat(j, b)

                b2 = (b + LEAD) % NB

                def _refill(j=j, b2=b2):
                    pltpu.make_async_copy(
                        gb[b2], acc.at[rowv.at[j - (NB - LEAD)]],
                        ssem[b2]).wait()
                    _gather(j + LEAD, b2, u_hbm)

                if b < NB - LEAD:
                    _refill()
                else:
                    pl.when(g < ng - 1)(_refill)

        for j in range(NCH - NB, NCH):
            pltpu.make_async_copy(gb[j % NB], acc.at[rowv.at[j]],
                                  ssem[j % NB]).wait()

        plsc.subcore_barrier()

        @pl.when(c == 0)
        def _wr0():
            pltpu.sync_copy(acc.at[pl.ds(s * RPT, RPT)],
                            out_c0.at[pl.ds(s * RPT, RPT)])

        @pl.when(c == 1)
        def _wr1():
            pltpu.sync_copy(acc.at[pl.ds(s * RPT, RPT)],
                            out_c1.at[pl.ds(s * RPT, RPT)])

        if h == 0:
            @pl.loop(0, RPT // 128)
            def _rz(i):
                pltpu.sync_copy(zed_v, acc.at[pl.ds(s * RPT + i * 128, 128)])


_BR = 2048
_GRID = N_PAD // _BR

NH = N_PAD // 2
_BRH = _BR // 2

_row_spec = pl.BlockSpec((_BR, D), lambda i: (i, 0))
_w_spec = pl.BlockSpec((D, D), lambda i: (0, 0))
_pk_spec = pl.BlockSpec((_BRH, D), lambda i: (i, 0))

_f32 = jnp.float32


def _tc_prep_body(dp2_ref, dinvp_ref):
    deg = dp2_ref[0] + dp2_ref[1]
    dinv = jnp.where(deg > 0.0, lax.rsqrt(jnp.maximum(deg, 1e-12)), 0.0)
    lane = lax.broadcasted_iota(jnp.int32, (_BRH, D), 1)
    dinvp_ref[...] = jnp.where(lane < D2, dinv[:, 0:1], dinv[:, 1:2])


_tc_prep = pl.pallas_call(
    _tc_prep_body,
    grid=(_GRID,),
    in_specs=[pl.BlockSpec((NC, _BRH, 2), lambda i: (0, i, 0))],
    out_specs=_pk_spec,
    out_shape=jax.ShapeDtypeStruct((NH, D), _f32),
)


_e_spec = pl.BlockSpec((2 * D, D), lambda i: (0, 0))
_bp_spec = pl.BlockSpec((1, D), lambda i: (0, 0))


def _pk_mm(t0p, t1p, e0_ref, e1_ref):
    tcat = jnp.concatenate([t0p, t1p], axis=1)
    a0 = jnp.dot(tcat, e0_ref[...], preferred_element_type=_f32)
    a1 = jnp.dot(tcat, e1_ref[...], preferred_element_type=_f32)
    return a0, a1


_pk_shape = jax.ShapeDtypeStruct((NH, D), _f32)


def _u_first_body(x0_ref, x1_ref, dinvp_ref, u0_ref, u1_ref):
    dinvp = dinvp_ref[...]
    u0_ref[...] = dinvp * x0_ref[...]
    u1_ref[...] = dinvp * x1_ref[...]


_tc_u_first = pl.pallas_call(
    _u_first_body, grid=(_GRID,),
    in_specs=[_pk_spec] * 3,
    out_specs=[_pk_spec] * 2, out_shape=[_pk_shape] * 2)


def _relu_u_body(a0_ref, a1_ref, b0_ref, b1_ref, dinvp_ref,
                 h0_ref, h1_ref, u0_ref, u1_ref):
    h0p = jnp.maximum(a0_ref[...] + b0_ref[...], 0.0)
    h1p = jnp.maximum(a1_ref[...] + b1_ref[...], 0.0)
    dinvp = dinvp_ref[...]
    h0_ref[...] = h0p
    h1_ref[...] = h1p
    u0_ref[...] = dinvp * h0p
    u1_ref[...] = dinvp * h1p


_tc_relu_u = pl.pallas_call(
    _relu_u_body, grid=(_GRID,),
    in_specs=[_pk_spec, _pk_spec, _bp_spec, _bp_spec, _pk_spec],
    out_specs=[_pk_spec] * 4, out_shape=[_pk_shape] * 4)


def _make_tc_comb(with_prev):
    alpha = 2.0 if with_prev else 1.0

    def body(*refs):
        if with_prev:
            (p00_ref, p01_ref, p10_ref, p11_ref, tp0_ref, tp1_ref,
             dinvp_ref, t0_ref, t1_ref, u0_ref, u1_ref) = refs
        else:
            (p00_ref, p01_ref, p10_ref, p11_ref,
             dinvp_ref, t0_ref, t1_ref, u0_ref, u1_ref) = refs
        dinvp = dinvp_ref[...]
        t0p = -alpha * dinvp * (p00_ref[...] + p10_ref[...])
        t1p = -alpha * dinvp * (p01_ref[...] + p11_ref[...])
        if with_prev:
            t0p = t0p - tp0_ref[...]
            t1p = t1p - tp1_ref[...]
        t0_ref[...] = t0p
        t1_ref[...] = t1p
        u0_ref[...] = dinvp * t0p
        u1_ref[...] = dinvp * t1p

    n_in = 7 if with_prev else 5
    return pl.pallas_call(body, grid=(_GRID,),
                          in_specs=[_pk_spec] * n_in,
                          out_specs=[_pk_spec] * 4,
                          out_shape=[_pk_shape] * 4)


_tc_comb_k1 = _make_tc_comb(False)
_tc_comb_mid = _make_tc_comb(True)


def _make_tc_accum(init):
    def body(*refs):
        if init:
            t0_ref, t1_ref, e0_ref, e1_ref, o0_ref, o1_ref = refs
        else:
            t0_ref, t1_ref, e0_ref, e1_ref, a0_ref, a1_ref, \
                o0_ref, o1_ref = refs
        m0, m1 = _pk_mm(t0_ref[...], t1_ref[...], e0_ref, e1_ref)
        o0_ref[...] = m0 if init else a0_ref[...] + m0
        o1_ref[...] = m1 if init else a1_ref[...] + m1

    in_specs = [_pk_spec, _pk_spec, _e_spec, _e_spec]
    if not init:
        in_specs += [_pk_spec, _pk_spec]
    return pl.pallas_call(body, grid=(_GRID,), in_specs=in_specs,
                          out_specs=[_pk_spec] * 2,
                          out_shape=[_pk_shape] * 2)


_tc_accum_init = _make_tc_accum(True)
_tc_accum = _make_tc_accum(False)


def _tc_step_last_body(p00_ref, p01_ref, p10_ref, p11_ref, tp0_ref, tp1_ref,
                       dinvp_ref, e0_ref, e1_ref, a0_ref, a1_ref,
                       o0_ref, o1_ref):
    dinvp = dinvp_ref[...]
    t0p = -2.0 * dinvp * (p00_ref[...] + p10_ref[...]) - tp0_ref[...]
    t1p = -2.0 * dinvp * (p01_ref[...] + p11_ref[...]) - tp1_ref[...]
    m0, m1 = _pk_mm(t0p, t1p, e0_ref, e1_ref)
    o0_ref[...] = a0_ref[...] + m0
    o1_ref[...] = a1_ref[...] + m1


_tc_step_last = pl.pallas_call(
    _tc_step_last_body, grid=(_GRID,),
    in_specs=[_pk_spec] * 7 + [_e_spec, _e_spec, _pk_spec, _pk_spec],
    out_specs=[_pk_spec] * 2, out_shape=[_pk_shape] * 2)


def _tc_head_body(a0_ref, a1_ref, b0_ref, b1_ref, el_ref, bl_ref, y_ref):
    tcat = jnp.concatenate([a0_ref[...] + b0_ref[...],
                            a1_ref[...] + b1_ref[...]], axis=1)
    y_ref[...] = jnp.dot(tcat, el_ref[...],
                         preferred_element_type=_f32) + bl_ref[0, 0]


_tc_head = pl.pallas_call(
    _tc_head_body,
    grid=(_GRID,),
    in_specs=[_pk_spec, _pk_spec, _bp_spec, _bp_spec,
              pl.BlockSpec((2 * D, 2), lambda i: (0, 0)),
              pl.BlockSpec((1, 1), lambda i: (0, 0))],
    out_specs=pl.BlockSpec((_BRH, 2), lambda i: (i, 0)),
    out_shape=jax.ShapeDtypeStruct((NH, 2), _f32),
)


def _blkdiag2(m):
    z = jnp.zeros_like(m)
    return jnp.concatenate([jnp.concatenate([m, z], 1),
                            jnp.concatenate([z, m], 1)], 0)


def _expand_w(w):
    e0 = jnp.concatenate([_blkdiag2(w[:D2, :D2]), _blkdiag2(w[D2:, :D2])], 0)
    e1 = jnp.concatenate([_blkdiag2(w[:D2, D2:]), _blkdiag2(w[D2:, D2:])], 0)
    return e0, e1


def _pack_vec(v):
    return (jnp.concatenate([v[:D2], v[:D2]]).reshape(1, D),
            jnp.concatenate([v[D2:], v[D2:]]).reshape(1, D))


def kernel(x, edge_index, W1, b1, W2, b2, W3, b3, Wl, bl):
    row_r = edge_index[0].reshape(NW, NCH, CH)
    col_r = edge_index[1].reshape(NW, NCH, CH)

    xp = jnp.concatenate([x, jnp.zeros((N_PAD - N, D), jnp.float32)], axis=0)
    x0p = jnp.reshape(xp[:, :D2], (NH, D))
    x1p = jnp.reshape(xp[:, D2:], (NH, D))

    dp = _deg_sc(row_r)
    dinvp = _tc_prep(jnp.reshape(dp, (NC, NH, 2)))

    def _sc_view(a):
        return jnp.reshape(a, (N_PAD, D2))

    def _tc_view(a):
        return jnp.reshape(a, (NH, D))

    a0 = a1 = None
    bprev = None
    for li, (W, b) in enumerate(((W1, b1), (W2, b2), (W3, b3))):
        es = [_expand_w(W[k]) for k in range(K)]
        if li == 0:
            h0p, h1p = x0p, x1p
            u0p, u1p = _tc_u_first(x0p, x1p, dinvp)
        else:
            b0p, b1p = _pack_vec(bprev)
            h0p, h1p, u0p, u1p = _tc_relu_u(a0, a1, b0p, b1p, dinvp)
        a0, a1 = _tc_accum_init(h0p, h1p, *es[0])
        txs = [(h0p, h1p)]
        for k in range(1, K):
            ps = _prop_sc(_sc_view(u0p), _sc_view(u1p), col_r, row_r)
            pr = tuple(_tc_view(p) for p in ps)
            if k == 1:
                t0p, t1p, u0p, u1p = _tc_comb_k1(*pr, dinvp)
            elif k < K - 1:
                t0p, t1p, u0p, u1p = _tc_comb_mid(*pr, *txs[k - 2], dinvp)
            else:
                a0, a1 = _tc_step_last(*pr, *txs[k - 2], dinvp, *es[k],
                                       a0, a1)
            if k < K - 1:
                a0, a1 = _tc_accum(t0p, t1p, *es[k], a0, a1)
            txs.append((t0p, t1p))
        bprev = b

    el = jnp.concatenate([_blkdiag2(Wl[:D2]), _blkdiag2(Wl[D2:])], 0)
    b30p, b31p = _pack_vec(b3)
    yp = _tc_head(a0, a1, b30p, b31p, el, bl.reshape(1, 1))
    return jnp.reshape(yp, (N_PAD, 1))[:N]
